# baseline (speedup 1.0000x reference)
import functools

import jax
import jax.numpy as jnp
from jax import lax
from jax.experimental import pallas as pl
from jax.experimental.pallas import tpu as pltpu
from jax.experimental.pallas import tpu_sc as plsc

_VOCAB = 1000000
_EMBED = 64
_BATCH = 4096
_HIST = 50

_NC = 2
_NS = 16
_NW = _NC * _NS
_BCOL = _BATCH // _NW
_L = 16
_NG = _BCOL // _L

_mesh = plsc.VectorSubcoreMesh(core_axis_name="c", subcore_axis_name="s")


@functools.partial(
    pl.kernel,
    mesh=_mesh,
    out_type=jax.ShapeDtypeStruct((_HIST, _EMBED, _BATCH), jnp.float32),
    scratch_types=[
        pltpu.VMEM((_HIST, _BCOL), jnp.int32),
        pltpu.VMEM((_HIST, _BCOL), jnp.int32),
        pltpu.VMEM((_BCOL, 2 * _EMBED), jnp.float32),
        pltpu.VMEM((_BCOL, 2 * _EMBED), jnp.float32),
        pltpu.VMEM((_EMBED, _BCOL), jnp.float32),
        pltpu.VMEM((_EMBED, _BCOL), jnp.float32),
        pltpu.SemaphoreType.DMA,
        pltpu.SemaphoreType.DMA,
        pltpu.SemaphoreType.DMA,
        pltpu.SemaphoreType.DMA,
    ],
    compiler_params=pltpu.CompilerParams(use_tc_tiling_on_sc=False,
                                         needs_layout_passes=False),
)
def _gather_kernel(idx2_hbm, pb_hbm, table2_hbm, out_hbm, idx2_v, pb_v,
                   raw_a, raw_b, tr_a, tr_b, gsem_a, gsem_b, osem_a, osem_b):
    wid = lax.axis_index("s") * _NC + lax.axis_index("c")
    b0 = wid * _BCOL
    pltpu.sync_copy(idx2_hbm.at[:, pl.ds(b0, _BCOL)], idx2_v)
    pltpu.sync_copy(pb_hbm.at[:, pl.ds(b0, _BCOL)], pb_v)

    slots = [jnp.arange(_L, dtype=jnp.int32) + (g * _L) for g in range(_NG)]

    def _gather(h, raw, sem):
        return pltpu.async_copy(table2_hbm.at[idx2_v.at[h]], raw, sem)

    def _writeback(h, tr, sem):
        return pltpu.async_copy(tr, out_hbm.at[h, :, pl.ds(b0, _BCOL)], sem)

    def _drain_write(tr, sem):
        pltpu.make_async_copy(out_hbm.at[0, :, pl.ds(b0, _BCOL)], tr,
                              sem).wait()

    def _extract(h, raw, tr):
        pbs = [pb_v[h, pl.ds(g * _L, _L)] for g in range(_NG)]

        def jbody(j, carry):
            for g in range(_NG):
                vals = plsc.load_gather(raw, [slots[g], pbs[g] + j])
                tr[j, pl.ds(g * _L, _L)] = vals
            return carry

        lax.fori_loop(0, _EMBED, jbody, 0)

    ga = _gather(0, raw_a, gsem_a)

    def body(i, carry):
        ha = 2 * i
        hb = ha + 1

        gb = _gather(hb, raw_b, gsem_b)
        pltpu.make_async_copy(table2_hbm.at[idx2_v.at[ha]], raw_a,
                              gsem_a).wait()

        @pl.when(i > 0)
        def _():
            _drain_write(tr_a, osem_a)

        _extract(ha, raw_a, tr_a)
        _writeback(ha, tr_a, osem_a)

        @pl.when(i < _HIST // 2 - 1)
        def _():
            _gather(ha + 2, raw_a, gsem_a)

        pltpu.make_async_copy(table2_hbm.at[idx2_v.at[hb]], raw_b,
                              gsem_b).wait()

        @pl.when(i > 0)
        def _():
            _drain_write(tr_b, osem_b)

        _extract(hb, raw_b, tr_b)
        _writeback(hb, tr_b, osem_b)
        return carry

    lax.fori_loop(0, _HIST // 2, body, 0)
    _drain_write(tr_a, osem_a)
    _drain_write(tr_b, osem_b)


def kernel(indices, table):
    idxT = indices.T.astype(jnp.int32)
    idx2 = idxT >> 1
    pb = (idxT & 1) << 6
    table2 = table.reshape(_VOCAB // 2, 2 * _EMBED)
    out = _gather_kernel(idx2, pb, table2)
    return out.transpose(2, 0, 1)


# speedup vs baseline: 1.0574x; 1.0574x over previous
import functools

import jax
import jax.numpy as jnp
from jax import lax
from jax.experimental import pallas as pl
from jax.experimental.pallas import tpu as pltpu
from jax.experimental.pallas import tpu_sc as plsc

_VOCAB = 1000000
_EMBED = 64
_BATCH = 4096
_HIST = 50

_NC = 2
_NS = 16
_NW = _NC * _NS
_BCOL = _BATCH // _NW
_L = 16
_NG = _BCOL // _L

_mesh = plsc.VectorSubcoreMesh(core_axis_name="c", subcore_axis_name="s")


@functools.partial(
    pl.kernel,
    mesh=_mesh,
    out_type=jax.ShapeDtypeStruct((_HIST, _EMBED, _BATCH), jnp.float32),
    scratch_types=[
        pltpu.VMEM((_HIST, _BCOL), jnp.int32),
        pltpu.VMEM((_HIST, _BCOL), jnp.int32),
        pltpu.VMEM((_BCOL, 2 * _EMBED), jnp.float32),
        pltpu.VMEM((_BCOL, 2 * _EMBED), jnp.float32),
        pltpu.VMEM((_EMBED, _BCOL), jnp.float32),
        pltpu.VMEM((_EMBED, _BCOL), jnp.float32),
        pltpu.SemaphoreType.DMA,
        pltpu.SemaphoreType.DMA,
        pltpu.SemaphoreType.DMA,
        pltpu.SemaphoreType.DMA,
    ],
    compiler_params=pltpu.CompilerParams(use_tc_tiling_on_sc=True,
                                         needs_layout_passes=False),
)
def _gather_kernel(idx2_hbm, pb_hbm, table2_hbm, out_hbm, idx2_v, pb_v,
                   raw_a, raw_b, tr_a, tr_b, gsem_a, gsem_b, osem_a, osem_b):
    wid = lax.axis_index("s") * _NC + lax.axis_index("c")
    b0 = wid * _BCOL
    pltpu.sync_copy(idx2_hbm.at[:, pl.ds(b0, _BCOL)], idx2_v)
    pltpu.sync_copy(pb_hbm.at[:, pl.ds(b0, _BCOL)], pb_v)

    slots = [jnp.arange(_L, dtype=jnp.int32) + (g * _L) for g in range(_NG)]

    def _gather(h, raw, sem):
        return pltpu.async_copy(table2_hbm.at[idx2_v.at[h]], raw, sem)

    def _writeback(h, tr, sem):
        return pltpu.async_copy(tr, out_hbm.at[h, :, pl.ds(b0, _BCOL)], sem)

    def _drain_write(tr, sem):
        pltpu.make_async_copy(out_hbm.at[0, :, pl.ds(b0, _BCOL)], tr,
                              sem).wait()

    def _extract(h, raw, tr):
        pbs = [pb_v[h, pl.ds(g * _L, _L)] for g in range(_NG)]

        def jbody(j, carry):
            for g in range(_NG):
                vals = plsc.load_gather(raw, [slots[g], pbs[g] + j])
                tr[j, pl.ds(g * _L, _L)] = vals
            return carry

        lax.fori_loop(0, _EMBED, jbody, 0)

    ga = _gather(0, raw_a, gsem_a)

    def body(i, carry):
        ha = 2 * i
        hb = ha + 1

        gb = _gather(hb, raw_b, gsem_b)
        pltpu.make_async_copy(table2_hbm.at[idx2_v.at[ha]], raw_a,
                              gsem_a).wait()

        @pl.when(i > 0)
        def _():
            _drain_write(tr_a, osem_a)

        _extract(ha, raw_a, tr_a)
        _writeback(ha, tr_a, osem_a)

        @pl.when(i < _HIST // 2 - 1)
        def _():
            _gather(ha + 2, raw_a, gsem_a)

        pltpu.make_async_copy(table2_hbm.at[idx2_v.at[hb]], raw_b,
                              gsem_b).wait()

        @pl.when(i > 0)
        def _():
            _drain_write(tr_b, osem_b)

        _extract(hb, raw_b, tr_b)
        _writeback(hb, tr_b, osem_b)
        return carry

    lax.fori_loop(0, _HIST // 2, body, 0)
    _drain_write(tr_a, osem_a)
    _drain_write(tr_b, osem_b)


def kernel(indices, table):
    idxT = indices.T.astype(jnp.int32)
    idx2 = idxT >> 1
    pb = (idxT & 1) << 6
    table2 = table.reshape(_VOCAB // 2, 2 * _EMBED)
    out = _gather_kernel(idx2, pb, table2)
    return out.transpose(2, 0, 1)


# speedup vs baseline: 1.2694x; 1.2004x over previous
import functools

import jax
import jax.numpy as jnp
from jax import lax
from jax.experimental import pallas as pl
from jax.experimental.pallas import tpu as pltpu
from jax.experimental.pallas import tpu_sc as plsc

_VOCAB = 1000000
_EMBED = 64
_BATCH = 4096
_HIST = 50

_B_TOTAL = _BATCH * _HIST
_NC = 2
_NS = 16
_NW = _NC * _NS
_B_PER_W = _B_TOTAL // _NW
_CHUNK = 640
_N_CHUNKS = _B_PER_W // _CHUNK
_N_PAIRS = _N_CHUNKS // 2

_mesh = plsc.VectorSubcoreMesh(core_axis_name="c", subcore_axis_name="s")


@functools.partial(
    pl.kernel,
    mesh=_mesh,
    out_type=jax.ShapeDtypeStruct((_B_TOTAL, _EMBED), jnp.float32),
    scratch_types=[
        pltpu.VMEM((_N_CHUNKS, _CHUNK), jnp.int32),
        pltpu.VMEM((_CHUNK, _EMBED), jnp.float32),
        pltpu.VMEM((_CHUNK, _EMBED), jnp.float32),
        pltpu.SemaphoreType.DMA,
        pltpu.SemaphoreType.DMA,
        pltpu.SemaphoreType.DMA,
        pltpu.SemaphoreType.DMA,
    ],
    compiler_params=pltpu.CompilerParams(use_tc_tiling_on_sc=False),
)
def _gather_kernel(idx_hbm, table_hbm, out_hbm, idx_v, buf_a, buf_b,
                   gsem_a, gsem_b, osem_a, osem_b):
    wid = lax.axis_index("s") * _NC + lax.axis_index("c")
    pltpu.sync_copy(idx_hbm.at[wid], idx_v)
    out_base = wid * _B_PER_W

    def _gather(c, buf, sem):
        return pltpu.async_copy(table_hbm.at[idx_v.at[c]], buf, sem)

    def _writeback(c, buf, sem):
        return pltpu.async_copy(
            buf, out_hbm.at[pl.ds(out_base + c * _CHUNK, _CHUNK)], sem)

    def _drain_write(buf, sem):
        pltpu.make_async_copy(out_hbm.at[pl.ds(out_base, _CHUNK)], buf,
                              sem).wait()

    def body(g, carry):
        ca = 2 * g
        cb = ca + 1

        ga = _gather(ca, buf_a, gsem_a)

        @pl.when(g > 0)
        def _():
            _drain_write(buf_b, osem_b)

        ga.wait()
        _writeback(ca, buf_a, osem_a)
        gb = _gather(cb, buf_b, gsem_b)

        @pl.when(g < _N_PAIRS - 1)
        def _():
            _drain_write(buf_a, osem_a)

        gb.wait()
        _writeback(cb, buf_b, osem_b)
        return carry

    lax.fori_loop(0, _N_PAIRS, body, 0)
    _drain_write(buf_a, osem_a)
    _drain_write(buf_b, osem_b)


def kernel(indices, table):
    idx = indices.T.astype(jnp.int32).reshape(_NW, _N_CHUNKS, _CHUNK)
    out = _gather_kernel(idx, table)
    return out.reshape(_HIST, _BATCH, _EMBED).transpose(1, 0, 2)


# speedup vs baseline: 1.7780x; 1.4007x over previous
import functools

import jax
import jax.numpy as jnp
from jax import lax
from jax.experimental import pallas as pl
from jax.experimental.pallas import tpu as pltpu
from jax.experimental.pallas import tpu_sc as plsc

_VOCAB = 1000000
_EMBED = 64
_BATCH = 4096
_HIST = 50

_B_TOTAL = _BATCH * _HIST
_NC = 2
_NS = 16
_NW = _NC * _NS
_B_PER_W = _B_TOTAL // _NW
_CHUNK = 128
_N_CHUNKS = _B_PER_W // _CHUNK
_N_PAIRS = _N_CHUNKS // 2

_HALF = 524288
_TBK = 4096
_TGRID = _HALF // _TBK
_LASTB = -(-_VOCAB // _TBK) - 1

_mesh = plsc.VectorSubcoreMesh(core_axis_name="c", subcore_axis_name="s")


def _pack_body(lo_ref, hi_ref, o_ref):
    o_ref[:, 0:_EMBED] = lo_ref[...].T
    o_ref[:, _EMBED:2 * _EMBED] = hi_ref[...].T


_pack = pl.pallas_call(
    _pack_body,
    grid=(_TGRID,),
    in_specs=[
        pl.BlockSpec((_EMBED, _TBK), lambda i: (0, i)),
        pl.BlockSpec((_EMBED, _TBK), lambda i: (0, jnp.minimum(i + _TGRID,
                                                               _LASTB))),
    ],
    out_specs=pl.BlockSpec((_TBK, 2 * _EMBED), lambda i: (i, 0)),
    out_shape=jax.ShapeDtypeStruct((_HALF, 2 * _EMBED), jnp.float32),
)


@functools.partial(
    pl.kernel,
    mesh=_mesh,
    out_type=jax.ShapeDtypeStruct((_B_TOTAL, 2 * _EMBED), jnp.float32),
    scratch_types=[
        pltpu.VMEM((_N_CHUNKS, _CHUNK), jnp.int32),
        pltpu.VMEM((_CHUNK, 2 * _EMBED), jnp.float32),
        pltpu.VMEM((_CHUNK, 2 * _EMBED), jnp.float32),
        pltpu.SemaphoreType.DMA,
        pltpu.SemaphoreType.DMA,
        pltpu.SemaphoreType.DMA,
        pltpu.SemaphoreType.DMA,
    ],
    compiler_params=pltpu.CompilerParams(use_tc_tiling_on_sc=True),
)
def _gather_kernel(idx_hbm, table_hbm, out_hbm, idx_v, buf_a, buf_b,
                   gsem_a, gsem_b, osem_a, osem_b):
    wid = lax.axis_index("s") * _NC + lax.axis_index("c")
    pltpu.sync_copy(idx_hbm.at[wid], idx_v)
    out_base = wid * _B_PER_W

    def _gather(c, buf, sem):
        return pltpu.async_copy(table_hbm.at[idx_v.at[c]], buf, sem)

    def _writeback(c, buf, sem):
        return pltpu.async_copy(
            buf, out_hbm.at[pl.ds(out_base + c * _CHUNK, _CHUNK)], sem)

    def _drain_write(buf, sem):
        pltpu.make_async_copy(out_hbm.at[pl.ds(out_base, _CHUNK)], buf,
                              sem).wait()

    def body(g, carry):
        ca = 2 * g
        cb = ca + 1

        ga = _gather(ca, buf_a, gsem_a)

        @pl.when(g > 0)
        def _():
            _drain_write(buf_b, osem_b)

        ga.wait()
        _writeback(ca, buf_a, osem_a)
        gb = _gather(cb, buf_b, gsem_b)

        @pl.when(g < _N_PAIRS - 1)
        def _():
            _drain_write(buf_a, osem_a)

        gb.wait()
        _writeback(cb, buf_b, osem_b)
        return carry

    lax.fori_loop(0, _N_PAIRS, body, 0)
    _drain_write(buf_a, osem_a)
    _drain_write(buf_b, osem_b)


def kernel(indices, table):
    flat = indices.T.astype(jnp.int32).reshape(_B_TOTAL)
    idx2 = (flat & (_HALF - 1)).reshape(_NW, _N_CHUNKS, _CHUNK)
    hi = (flat >= _HALF)[:, None]
    tT = table.T
    table_pack = _pack(tT, tT)
    out128 = _gather_kernel(idx2, table_pack)
    out = jnp.where(hi, out128[:, _EMBED:2 * _EMBED], out128[:, 0:_EMBED])
    return out.reshape(_HIST, _BATCH, _EMBED).transpose(1, 0, 2)
